# Initial kernel scaffold; baseline (speedup 1.0000x reference)
import jax
import jax.numpy as jnp
from jax.experimental import pallas as pl


def kernel(boxes, scores, class_ids):
    raise NotImplementedError("write your pallas kernel here")


# speedup vs baseline: 68.3049x; 68.3049x over previous
import jax
import jax.numpy as jnp
from jax.experimental import pallas as pl
from jax.experimental.pallas import tpu as pltpu

_NP = 5120
_BLK = 128
_NBLK = _NP // _BLK
_IOU_THRESHOLD = 0.5


def _nms_kernel(data_ref, data_t_ref, out_ref, d_ref):
    x1 = data_ref[0:1, :]
    y1 = data_ref[1:2, :]
    x2 = data_ref[2:3, :]
    y2 = data_ref[3:4, :]
    s = data_ref[4:5, :]
    cls = data_ref[5:6, :]
    v = data_ref[6:7, :]

    cmax = jnp.maximum(jnp.maximum(x1, x2), jnp.maximum(y1, y2))
    cmax = jnp.where(v > 0, cmax, -jnp.inf)
    m = jnp.max(cmax)

    off = cls * (m + 1.0)
    ox1 = x1 + off
    oy1 = y1 + off
    ox2 = x2 + off
    oy2 = y2 + off
    area = (ox2 - ox1) * (oy2 - oy1)

    ia = jax.lax.broadcasted_iota(jnp.int32, (1, _NP), 1)

    def build(i, carry):
        r = i * _BLK
        rows = data_t_ref[pl.ds(r, _BLK), :]
        bx1 = rows[:, 0:1]
        by1 = rows[:, 1:2]
        bx2 = rows[:, 2:3]
        by2 = rows[:, 3:4]
        bs = rows[:, 4:5]
        bcls = rows[:, 5:6]
        bv = rows[:, 6:7]
        boff = bcls * (m + 1.0)
        bx1 = bx1 + boff
        by1 = by1 + boff
        bx2 = bx2 + boff
        by2 = by2 + boff
        barea = (bx2 - bx1) * (by2 - by1)

        ix1 = jnp.maximum(bx1, ox1)
        iy1 = jnp.maximum(by1, oy1)
        ix2 = jnp.minimum(bx2, ox2)
        iy2 = jnp.minimum(by2, oy2)
        w = jnp.maximum(ix2 - ix1, 0.0)
        h = jnp.maximum(iy2 - iy1, 0.0)
        inter = w * h
        union = barea + area - inter
        iou_gt = inter > _IOU_THRESHOLD * jnp.maximum(union, 1e-9)

        ib = jax.lax.broadcasted_iota(jnp.int32, (_BLK, 1), 0) + r
        dom = (bs > s) | ((bs == s) & (ib < ia))
        blk = iou_gt & dom & (bv > 0) & (v > 0)
        d_ref[pl.ds(r, _BLK), :] = blk.astype(jnp.int8)
        return carry

    jax.lax.fori_loop(0, _NBLK, build, 0)

    keep0 = jnp.where(v > 0, 1.0, 0.0).astype(jnp.float32)

    def cond(carry):
        changed, _ = carry
        return changed

    def body(carry):
        _, keep = carry
        ki8 = keep.astype(jnp.int8)
        supp = jax.lax.dot_general(
            ki8, d_ref[...],
            dimension_numbers=(((1,), (0,)), ((), ())),
            preferred_element_type=jnp.int32,
        )
        nk = jnp.where((supp == 0) & (v > 0), 1.0, 0.0).astype(jnp.float32)
        return jnp.any(nk != keep), nk

    _, keep = jax.lax.while_loop(cond, body, (jnp.asarray(True), keep0))

    out_ref[...] = jnp.concatenate(
        [keep, keep * s, jnp.zeros((6, _NP), jnp.float32)], axis=0)


def kernel(boxes, scores, class_ids):
    n = boxes.shape[0]
    x1 = boxes[:, 0]
    y1 = boxes[:, 1]
    x2 = boxes[:, 2]
    y2 = boxes[:, 3]
    data = jnp.stack(
        [x1, y1, x2, y2, scores,
         class_ids.astype(jnp.float32),
         jnp.ones((n,), jnp.float32),
         jnp.zeros((n,), jnp.float32)], axis=0)
    data = jnp.pad(data, ((0, 0), (0, _NP - n)))
    data_t = data.T

    out = pl.pallas_call(
        _nms_kernel,
        out_shape=jax.ShapeDtypeStruct((8, _NP), jnp.float32),
        scratch_shapes=[pltpu.VMEM((_NP, _NP), jnp.int8)],
    )(data, data_t)

    keep_mask = out[0, :n]
    kept_scores = out[1, :n]
    return (keep_mask, kept_scores)
